# baseline (speedup 1.0000x reference)
import functools

import jax
import jax.numpy as jnp
from jax import lax
from jax.experimental import pallas as pl
from jax.experimental.pallas import tpu as pltpu

HID = 256
N_OCC = 50000
N_CH = 1000
N_SEC = 4000
NUM_CLASSES = 1001


def _combine_body(m_ref, cnt_ref, h_ref, wl_ref, wr_ref, bias_ref, g_ref,
                  b_ref, out_ref, *, T, div):
    h = h_ref[...]
    acc = jnp.dot(h, wr_ref[...], preferred_element_type=jnp.float32)
    for t in range(T):
        inv = 1.0 / jnp.maximum(cnt_ref[:, t][:, None], 1.0)
        acc += jnp.dot(m_ref[t] * inv, wl_ref[t],
                       preferred_element_type=jnp.float32)
    acc = (acc + bias_ref[...]) * (1.0 / div)
    mu = jnp.mean(acc, axis=-1, keepdims=True)
    var = jnp.mean((acc - mu) ** 2, axis=-1, keepdims=True)
    y = (acc - mu) * lax.rsqrt(var + 1e-5) * g_ref[...] + b_ref[...]
    out_ref[...] = y + h


def _combine(m_stack, cnt, h, wl_stack, wr_sum, bias_sum, g, b, div, rb):
    T, N, _ = m_stack.shape
    grid = (N // rb,)
    return pl.pallas_call(
        functools.partial(_combine_body, T=T, div=div),
        grid=grid,
        in_specs=[
            pl.BlockSpec((T, rb, HID), lambda i: (0, i, 0)),
            pl.BlockSpec((rb, 8), lambda i: (i, 0)),
            pl.BlockSpec((rb, HID), lambda i: (i, 0)),
            pl.BlockSpec((T, HID, HID), lambda i: (0, 0, 0)),
            pl.BlockSpec((HID, HID), lambda i: (0, 0)),
            pl.BlockSpec((1, HID), lambda i: (0, 0)),
            pl.BlockSpec((1, HID), lambda i: (0, 0)),
            pl.BlockSpec((1, HID), lambda i: (0, 0)),
        ],
        out_specs=pl.BlockSpec((rb, HID), lambda i: (i, 0)),
        out_shape=jax.ShapeDtypeStruct((N, HID), jnp.float32),
    )(m_stack, cnt, h, wl_stack, wr_sum, bias_sum.reshape(1, HID),
      g.reshape(1, HID), b.reshape(1, HID))


def _matmul_bias_body(x_ref, w_ref, b_ref, out_ref):
    out_ref[...] = (
        jnp.dot(x_ref[...], w_ref[...], preferred_element_type=jnp.float32)
        + b_ref[...])


def _matmul_bias(x, w, b, rb, cb):
    N, K = x.shape
    M = w.shape[1]
    grid = (N // rb, pl.cdiv(M, cb))
    return pl.pallas_call(
        _matmul_bias_body,
        grid=grid,
        in_specs=[
            pl.BlockSpec((rb, K), lambda i, j: (i, 0)),
            pl.BlockSpec((K, cb), lambda i, j: (0, j)),
            pl.BlockSpec((1, cb), lambda i, j: (0, j)),
        ],
        out_specs=pl.BlockSpec((rb, cb), lambda i, j: (i, j)),
        out_shape=jax.ShapeDtypeStruct((N, M), jnp.float32),
    )(x, w, b.reshape(1, M))


def _seg_sum_cnt(src_tab, ei, n_dst):
    msg = src_tab[ei[0]]
    s = jax.ops.segment_sum(msg, ei[1], num_segments=n_dst)
    c = jax.ops.segment_sum(jnp.ones((ei.shape[1],), jnp.float32), ei[1],
                            num_segments=n_dst)
    return s, c


def kernel(x_occ, x_chord, x_sec, ei_next, ei_prev, ei_instance_of,
           ei_inst_rev, ei_in_section, ei_sec_rev, ei_next_section,
           Wp_occ, bp_occ, Wp_ch, bp_ch, Wp_sec, bp_sec,
           Wl, bl, Wr, ln_g, ln_b, Wc, bc):
    cf = jnp.zeros((N_OCC, Wp_ch.shape[0]), x_occ.dtype)
    cf = cf.at[ei_inst_rev[1]].set(x_chord[ei_inst_rev[0]])
    occ_in = jnp.concatenate([x_occ, cf], axis=1)
    h_occ = _matmul_bias(occ_in, Wp_occ, bp_occ, 1000, HID)
    h_ch = _matmul_bias(x_chord, Wp_ch, bp_ch, 1000, HID)
    h_sec = _matmul_bias(x_sec, Wp_sec, bp_sec, 1000, HID)

    for l in range(2):
        m0, c0 = _seg_sum_cnt(h_occ, ei_next, N_OCC)
        m1, c1 = _seg_sum_cnt(h_occ, ei_prev, N_OCC)
        m2, c2 = _seg_sum_cnt(h_occ, ei_instance_of, N_CH)
        m3, c3 = _seg_sum_cnt(h_ch, ei_inst_rev, N_OCC)
        m4, c4 = _seg_sum_cnt(h_occ, ei_in_section, N_SEC)
        m5, c5 = _seg_sum_cnt(h_sec, ei_sec_rev, N_OCC)
        m6, c6 = _seg_sum_cnt(h_sec, ei_next_section, N_SEC)

        occ_m = jnp.stack([m0, m1, m3, m5])
        occ_c = jnp.stack([c0, c1, c3, c5] + [c0] * 4, axis=1)
        occ_wl = jnp.stack([Wl[l, 0], Wl[l, 1], Wl[l, 3], Wl[l, 5]])
        occ_wr = Wr[l, 0] + Wr[l, 1] + Wr[l, 3] + Wr[l, 5]
        occ_bl = bl[l, 0] + bl[l, 1] + bl[l, 3] + bl[l, 5]
        new_occ = _combine(occ_m, occ_c, h_occ, occ_wl, occ_wr, occ_bl,
                           ln_g[l], ln_b[l], 4.0, 1000)

        ch_m = jnp.stack([m2])
        ch_c = jnp.stack([c2] * 8, axis=1)
        new_ch = _combine(ch_m, ch_c, h_ch, Wl[l, 2][None], Wr[l, 2],
                          bl[l, 2], ln_g[l], ln_b[l], 1.0, 1000)

        sec_m = jnp.stack([m4, m6])
        sec_c = jnp.stack([c4, c6] + [c4] * 6, axis=1)
        sec_wl = jnp.stack([Wl[l, 4], Wl[l, 6]])
        new_sec = _combine(sec_m, sec_c, h_sec, sec_wl, Wr[l, 4] + Wr[l, 6],
                           bl[l, 4] + bl[l, 6], ln_g[l], ln_b[l], 2.0, 1000)

        h_occ, h_ch, h_sec = new_occ, new_ch, new_sec

    return _matmul_bias(h_occ, Wc, bc, 1000, 256)
